# baseline (speedup 1.0000x reference)
import functools

import jax
import jax.numpy as jnp
from jax import lax
from jax.experimental import pallas as pl
from jax.experimental.pallas import tpu as pltpu
from jax.experimental.pallas import tpu_sc as plsc

_NEG = 5
_EPS = 1e-5
_D = 128
_L = 16

_E_POS = 100000
_P_ROWS = 784
_P_SEG = _P_ROWS * 128
_N_SEG = _P_SEG * _NEG
_N_ROWS = _N_SEG // 128
_NW = 32
_PPW = _P_SEG // _NW
_NPW = _PPW * _NEG
_PC = 16
_NC_ = _PC * _NEG
_NCH = _PPW // _PC


_CR = 192


def _sc_body(h_hbm, edges_hbm, out_hbm, mrr_hbm,
             idx_all, rows0, rows1,
             mat_d, mat_u, mat_v, negbuf, out_p, out_n,
             sem0, sem1):
  nc = 2
  wid = lax.axis_index("s") * nc + lax.axis_index("c")
  pbase = wid * _PPW
  nbase = _P_SEG + wid * _NPW

  pltpu.sync_copy(edges_hbm.at[pl.ds(wid * _NCH * _CR, _NCH * _CR)], idx_all)

  lane = lax.broadcasted_iota(jnp.int32, (_L,), 0)

  def bufs(slot):
    return (rows0, sem0) if slot == 0 else (rows1, sem1)

  def gather(t, slot):
    rows, sem = bufs(slot)
    pltpu.async_copy(h_hbm.at[idx_all.at[pl.ds(t * _CR, _CR)]], rows, sem)

  def wait(t, slot):
    rows, sem = bufs(slot)
    pltpu.make_async_copy(h_hbm.at[idx_all.at[pl.ds(t * _CR, _CR)]], rows,
                          sem).wait()

  def _tree(xs):
    while len(xs) > 1:
      xs = [xs[i] + xs[i + 1] for i in range(0, len(xs), 2)]
    return xs[0]

  def gamma_group(rows, u0, v0):
    for jj in range(_L):
      accu = jnp.zeros((_L,), jnp.float32)
      accv = jnp.zeros((_L,), jnp.float32)
      accw = jnp.zeros((_L,), jnp.float32)
      for c in range(_D // _L):
        u = rows[u0 + jj, pl.ds(c * _L, _L)]
        v = rows[v0 + jj, pl.ds(c * _L, _L)]
        accu = accu + u * u
        accv = accv + v * v
        accw = accw + u * v
      mat_u[jj, pl.ds(0, _L)] = accu
      mat_v[jj, pl.ds(0, _L)] = accv
      mat_d[jj, pl.ds(0, _L)] = accw
    cols = [jnp.full((_L,), c, jnp.int32) for c in range(_L)]
    gu = _tree([plsc.load_gather(mat_u, [lane, col]) for col in cols])
    gv = _tree([plsc.load_gather(mat_v, [lane, col]) for col in cols])
    gw = _tree([plsc.load_gather(mat_d, [lane, col]) for col in cols])
    gd = gu + gv - 2.0 * gw
    alpha = jnp.maximum(1.0 - gu, _EPS)
    beta = jnp.maximum(1.0 - gv, _EPS)
    return jnp.maximum(1.0 + 2.0 * gd / (alpha * beta), 1.0 + _EPS)

  def compute(t, slot, mrr_acc):
    rows = bufs(slot)[0]
    gp = gamma_group(rows, 0, _PC)
    out_p[pl.ds(t * _PC, _L)] = gp
    for g in range(_NEG):
      gn = gamma_group(rows, 2 * _PC + g * _L, 2 * _PC + _NC_ + g * _L)
      out_n[pl.ds(t * _NC_ + g * _L, _L)] = gn
      negbuf[pl.ds(g * _L, _L)] = gn
    rank = jnp.zeros((_L,), jnp.float32)
    for k in range(_NEG):
      gnk = plsc.load_gather(negbuf, [lane * _NEG + k])
      rank = rank + jnp.where(gnk <= gp, 1.0, 0.0)
    gpos = pbase + t * _PC + lane
    contrib = jnp.where(gpos < _E_POS, 1.0 / (rank + 1.0), 0.0)
    return mrr_acc + contrib

  gather(0, 0)

  def pair_body(p, mrr_acc):
    t0 = 2 * p
    wait(t0, 0)
    gather(t0 + 1, 1)
    mrr_acc = compute(t0, 0, mrr_acc)

    @pl.when(t0 + 2 < _NCH)
    def _():
      gather(t0 + 2, 0)

    wait(t0 + 1, 1)
    mrr_acc = compute(t0 + 1, 1, mrr_acc)
    return mrr_acc

  mrr_acc = lax.fori_loop(0, _NCH // 2, pair_body,
                          jnp.zeros((_L,), jnp.float32), unroll=False)

  negbuf[pl.ds(0, _L)] = mrr_acc
  pltpu.sync_copy(out_p, out_hbm.at[pl.ds(pbase, _PPW)])
  pltpu.sync_copy(out_n, out_hbm.at[pl.ds(nbase, _NPW)])
  pltpu.sync_copy(negbuf.at[pl.ds(0, _L)], mrr_hbm.at[pl.ds(wid * _L, _L)])


@jax.jit
def _sc_gamma(h, edges):
  mesh = plsc.VectorSubcoreMesh(core_axis_name="c", subcore_axis_name="s")
  k = pl.kernel(
      _sc_body,
      out_type=(jax.ShapeDtypeStruct((_P_SEG + _N_SEG,), jnp.float32),
                jax.ShapeDtypeStruct((_NW * _L,), jnp.float32)),
      mesh=mesh,
      compiler_params=pltpu.CompilerParams(needs_layout_passes=False),
      scratch_types=[
          pltpu.VMEM((_NCH * _CR,), jnp.int32),
          pltpu.VMEM((_CR, _D), jnp.float32),
          pltpu.VMEM((_CR, _D), jnp.float32),
          pltpu.VMEM((_L, _L + 1), jnp.float32),
          pltpu.VMEM((_L, _L + 1), jnp.float32),
          pltpu.VMEM((_L, _L + 1), jnp.float32),
          pltpu.VMEM((_NC_,), jnp.float32),
          pltpu.VMEM((_PPW,), jnp.float32),
          pltpu.VMEM((_NPW,), jnp.float32),
          pltpu.SemaphoreType.DMA,
          pltpu.SemaphoreType.DMA,
      ],
  )
  return k(h, edges)


def _dist2(g):
  g = jnp.maximum(g, 1.0 + _EPS)
  a = jnp.log(g + jnp.sqrt((g - 1.0) * (g + 1.0)))
  return a * a


def _tc_body(gp_ref, gn_ref, mrr_ref, out_ref):
  prow = lax.broadcasted_iota(jnp.int32, (_P_ROWS, 128), 0)
  pcol = lax.broadcasted_iota(jnp.int32, (_P_ROWS, 128), 1)
  pvalid = (prow * 128 + pcol) < _E_POS
  sp = _dist2(gp_ref[...])
  pos_sum = jnp.sum(jnp.where(pvalid, jnp.log(jax.nn.sigmoid(-sp) + 1e-5), 0.0))

  nrow = lax.broadcasted_iota(jnp.int32, (_N_ROWS, 128), 0)
  ncol = lax.broadcasted_iota(jnp.int32, (_N_ROWS, 128), 1)
  nvalid = (nrow * 128 + ncol) < (_E_POS * _NEG)
  sn = _dist2(gn_ref[...])
  neg_sum = jnp.sum(jnp.where(nvalid, jnp.log(jax.nn.sigmoid(sn) + 1e-5), 0.0))

  mrr = jnp.sum(mrr_ref[...]) / _E_POS
  loss = -(pos_sum / _E_POS) - (neg_sum / (_NEG * _E_POS))
  out_ref[...] = jnp.concatenate(
      [jnp.full((1, 128), loss, jnp.float32),
       jnp.full((1, 128), mrr, jnp.float32)], axis=0)


@jax.jit
def _tc_loss(gp2, gn2, mrr2):
  out = pl.pallas_call(
      _tc_body,
      out_shape=jax.ShapeDtypeStruct((2, 128), jnp.float32),
  )(gp2, gn2, mrr2)
  return out[0, 0], out[1, 0]


def kernel(h, pos_src, pos_dst, neg_src, neg_dst):
  ppad = _P_SEG - _E_POS
  npad = _N_SEG - _E_POS * _NEG
  a = jnp.pad(pos_src, (0, ppad)).reshape(_NW, _NCH, _PC)
  b = jnp.pad(pos_dst, (0, ppad)).reshape(_NW, _NCH, _PC)
  c = jnp.pad(neg_src, (0, npad)).reshape(_NW, _NCH, _NC_)
  d = jnp.pad(neg_dst, (0, npad)).reshape(_NW, _NCH, _NC_)
  edges = jnp.concatenate([a, b, c, d], axis=2).reshape(-1)

  gamma, mrr_parts = _sc_gamma(h, edges)
  gp2 = gamma[:_P_SEG].reshape(_P_ROWS, 128)
  gn2 = gamma[_P_SEG:].reshape(_N_ROWS, 128)
  loss, mrr = _tc_loss(gp2, gn2, mrr_parts.reshape(4, 128))
  return (loss, mrr)


# speedup vs baseline: 1.0179x; 1.0179x over previous
import functools

import jax
import jax.numpy as jnp
from jax import lax
from jax.experimental import pallas as pl
from jax.experimental.pallas import tpu as pltpu
from jax.experimental.pallas import tpu_sc as plsc

_NEG = 5
_EPS = 1e-5
_D = 128
_L = 16

_E_POS = 100000
_P_ROWS = 784
_P_SEG = _P_ROWS * 128
_N_SEG = _P_SEG * _NEG
_N_ROWS = _N_SEG // 128
_NW = 32
_PPW = _P_SEG // _NW
_NPW = _PPW * _NEG
_PC = 16
_NC_ = _PC * _NEG
_NCH = _PPW // _PC


def _sc_body(h_hbm, src_hbm, dst_hbm, out_hbm, mrr_hbm,
             idx_pu, idx_pv, idx_nu, idx_nv,
             pu0, pv0, nu0, nv0, pu1, pv1, nu1, nv1,
             mat_d, mat_u, mat_v, negbuf, out_p, out_n,
             spu0, spv0, snu0, snv0, spu1, spv1, snu1, snv1):
  nc = 2
  wid = lax.axis_index("s") * nc + lax.axis_index("c")
  pbase = wid * _PPW
  nbase = _P_SEG + wid * _NPW

  pltpu.sync_copy(src_hbm.at[pl.ds(pbase, _PPW)], idx_pu)
  pltpu.sync_copy(dst_hbm.at[pl.ds(pbase, _PPW)], idx_pv)
  pltpu.sync_copy(src_hbm.at[pl.ds(nbase, _NPW)], idx_nu)
  pltpu.sync_copy(dst_hbm.at[pl.ds(nbase, _NPW)], idx_nv)

  lane = lax.broadcasted_iota(jnp.int32, (_L,), 0)

  def bufs(slot):
    return ((pu0, pv0, nu0, nv0, spu0, spv0, snu0, snv0) if slot == 0 else
            (pu1, pv1, nu1, nv1, spu1, spv1, snu1, snv1))

  def gather(t, slot):
    pu, pv, nu, nv, spu, spv, snu, snv = bufs(slot)
    pltpu.async_copy(h_hbm.at[idx_pu.at[pl.ds(t * _PC, _PC)]], pu, spu)
    pltpu.async_copy(h_hbm.at[idx_pv.at[pl.ds(t * _PC, _PC)]], pv, spv)
    pltpu.async_copy(h_hbm.at[idx_nu.at[pl.ds(t * _NC_, _NC_)]], nu, snu)
    pltpu.async_copy(h_hbm.at[idx_nv.at[pl.ds(t * _NC_, _NC_)]], nv, snv)

  def wait(t, slot):
    pu, pv, nu, nv, spu, spv, snu, snv = bufs(slot)
    pltpu.make_async_copy(h_hbm.at[idx_pu.at[pl.ds(t * _PC, _PC)]], pu, spu).wait()
    pltpu.make_async_copy(h_hbm.at[idx_pv.at[pl.ds(t * _PC, _PC)]], pv, spv).wait()
    pltpu.make_async_copy(h_hbm.at[idx_nu.at[pl.ds(t * _NC_, _NC_)]], nu, snu).wait()
    pltpu.make_async_copy(h_hbm.at[idx_nv.at[pl.ds(t * _NC_, _NC_)]], nv, snv).wait()

  def _tree(xs):
    while len(xs) > 1:
      xs = [xs[i] + xs[i + 1] for i in range(0, len(xs), 2)]
    return xs[0]

  def gamma_group(ru, rv, row0):
    for jj in range(_L):
      accu = jnp.zeros((_L,), jnp.float32)
      accv = jnp.zeros((_L,), jnp.float32)
      accw = jnp.zeros((_L,), jnp.float32)
      for c in range(_D // _L):
        u = ru[row0 + jj, pl.ds(c * _L, _L)]
        v = rv[row0 + jj, pl.ds(c * _L, _L)]
        accu = accu + u * u
        accv = accv + v * v
        accw = accw + u * v
      mat_u[jj, pl.ds(0, _L)] = accu
      mat_v[jj, pl.ds(0, _L)] = accv
      mat_d[jj, pl.ds(0, _L)] = accw
    cols = [jnp.full((_L,), c, jnp.int32) for c in range(_L)]
    gu = _tree([plsc.load_gather(mat_u, [lane, col]) for col in cols])
    gv = _tree([plsc.load_gather(mat_v, [lane, col]) for col in cols])
    gw = _tree([plsc.load_gather(mat_d, [lane, col]) for col in cols])
    gd = gu + gv - 2.0 * gw
    alpha = jnp.maximum(1.0 - gu, _EPS)
    beta = jnp.maximum(1.0 - gv, _EPS)
    return jnp.maximum(1.0 + 2.0 * gd / (alpha * beta), 1.0 + _EPS)

  def compute(t, slot, mrr_acc):
    pu, pv, nu, nv = bufs(slot)[:4]
    gp = gamma_group(pu, pv, 0)
    out_p[pl.ds(t * _PC, _L)] = gp
    for g in range(_NEG):
      gn = gamma_group(nu, nv, g * _L)
      out_n[pl.ds(t * _NC_ + g * _L, _L)] = gn
      negbuf[pl.ds(g * _L, _L)] = gn
    rank = jnp.zeros((_L,), jnp.float32)
    for k in range(_NEG):
      gnk = plsc.load_gather(negbuf, [lane * _NEG + k])
      rank = rank + jnp.where(gnk <= gp, 1.0, 0.0)
    gpos = pbase + t * _PC + lane
    contrib = jnp.where(gpos < _E_POS, 1.0 / (rank + 1.0), 0.0)
    return mrr_acc + contrib

  gather(0, 0)

  def pair_body(p, mrr_acc):
    t0 = 2 * p
    wait(t0, 0)
    gather(t0 + 1, 1)
    mrr_acc = compute(t0, 0, mrr_acc)

    @pl.when(t0 + 2 < _NCH)
    def _():
      gather(t0 + 2, 0)

    wait(t0 + 1, 1)
    mrr_acc = compute(t0 + 1, 1, mrr_acc)
    return mrr_acc

  mrr_acc = lax.fori_loop(0, _NCH // 2, pair_body,
                          jnp.zeros((_L,), jnp.float32), unroll=False)

  negbuf[pl.ds(0, _L)] = mrr_acc
  pltpu.sync_copy(out_p, out_hbm.at[pl.ds(pbase, _PPW)])
  pltpu.sync_copy(out_n, out_hbm.at[pl.ds(nbase, _NPW)])
  pltpu.sync_copy(negbuf.at[pl.ds(0, _L)], mrr_hbm.at[pl.ds(wid * _L, _L)])


@jax.jit
def _sc_gamma(h, src_all, dst_all):
  mesh = plsc.VectorSubcoreMesh(core_axis_name="c", subcore_axis_name="s")
  k = pl.kernel(
      _sc_body,
      out_type=(jax.ShapeDtypeStruct((_P_SEG + _N_SEG,), jnp.float32),
                jax.ShapeDtypeStruct((_NW * _L,), jnp.float32)),
      mesh=mesh,
      compiler_params=pltpu.CompilerParams(needs_layout_passes=False),
      scratch_types=[
          pltpu.VMEM((_PPW,), jnp.int32),
          pltpu.VMEM((_PPW,), jnp.int32),
          pltpu.VMEM((_NPW,), jnp.int32),
          pltpu.VMEM((_NPW,), jnp.int32),
          pltpu.VMEM((_PC, _D), jnp.float32),
          pltpu.VMEM((_PC, _D), jnp.float32),
          pltpu.VMEM((_NC_, _D), jnp.float32),
          pltpu.VMEM((_NC_, _D), jnp.float32),
          pltpu.VMEM((_PC, _D), jnp.float32),
          pltpu.VMEM((_PC, _D), jnp.float32),
          pltpu.VMEM((_NC_, _D), jnp.float32),
          pltpu.VMEM((_NC_, _D), jnp.float32),
          pltpu.VMEM((_L, _L + 1), jnp.float32),
          pltpu.VMEM((_L, _L + 1), jnp.float32),
          pltpu.VMEM((_L, _L + 1), jnp.float32),
          pltpu.VMEM((_NC_,), jnp.float32),
          pltpu.VMEM((_PPW,), jnp.float32),
          pltpu.VMEM((_NPW,), jnp.float32),
          pltpu.SemaphoreType.DMA,
          pltpu.SemaphoreType.DMA,
          pltpu.SemaphoreType.DMA,
          pltpu.SemaphoreType.DMA,
          pltpu.SemaphoreType.DMA,
          pltpu.SemaphoreType.DMA,
          pltpu.SemaphoreType.DMA,
          pltpu.SemaphoreType.DMA,
      ],
  )
  return k(h, src_all, dst_all)


def _dist2(g):
  g = jnp.maximum(g, 1.0 + _EPS)
  a = jnp.log(g + jnp.sqrt((g - 1.0) * (g + 1.0)))
  return a * a


def _tc_body(gp_ref, gn_ref, mrr_ref, out_ref):
  prow = lax.broadcasted_iota(jnp.int32, (_P_ROWS, 128), 0)
  pcol = lax.broadcasted_iota(jnp.int32, (_P_ROWS, 128), 1)
  pvalid = (prow * 128 + pcol) < _E_POS
  sp = _dist2(gp_ref[...])
  pos_sum = jnp.sum(jnp.where(pvalid, jnp.log(jax.nn.sigmoid(-sp) + 1e-5), 0.0))

  nrow = lax.broadcasted_iota(jnp.int32, (_N_ROWS, 128), 0)
  ncol = lax.broadcasted_iota(jnp.int32, (_N_ROWS, 128), 1)
  nvalid = (nrow * 128 + ncol) < (_E_POS * _NEG)
  sn = _dist2(gn_ref[...])
  neg_sum = jnp.sum(jnp.where(nvalid, jnp.log(jax.nn.sigmoid(sn) + 1e-5), 0.0))

  mrr = jnp.sum(mrr_ref[...]) / _E_POS
  loss = -(pos_sum / _E_POS) - (neg_sum / (_NEG * _E_POS))
  out_ref[...] = jnp.concatenate(
      [jnp.full((1, 128), loss, jnp.float32),
       jnp.full((1, 128), mrr, jnp.float32)], axis=0)


@jax.jit
def _tc_loss(gp2, gn2, mrr2):
  out = pl.pallas_call(
      _tc_body,
      out_shape=jax.ShapeDtypeStruct((2, 128), jnp.float32),
  )(gp2, gn2, mrr2)
  return out[0, 0], out[1, 0]


def kernel(h, pos_src, pos_dst, neg_src, neg_dst):
  ppad = _P_SEG - _E_POS
  npad = _N_SEG - _E_POS * _NEG
  src_all = jnp.concatenate([jnp.pad(pos_src, (0, ppad)),
                             jnp.pad(neg_src, (0, npad))])
  dst_all = jnp.concatenate([jnp.pad(pos_dst, (0, ppad)),
                             jnp.pad(neg_dst, (0, npad))])

  gamma, mrr_parts = _sc_gamma(h, src_all, dst_all)
  gp2 = gamma[:_P_SEG].reshape(_P_ROWS, 128)
  gn2 = gamma[_P_SEG:].reshape(_N_ROWS, 128)
  loss, mrr = _tc_loss(gp2, gn2, mrr_parts.reshape(4, 128))
  return (loss, mrr)
